# baseline (speedup 1.0000x reference)
import functools
import math

import jax
import jax.numpy as jnp
from jax.experimental import pallas as pl
from jax.experimental.pallas import tpu as pltpu

_PI = math.pi

_ACOS4_PI = tuple(
    c / _PI for c in (-0.0187293, 0.0742610, -0.2121144, 1.5707288))
_INV_PI = 1.0 / _PI


def _acospi_poly(x):
    p = jnp.float32(_ACOS4_PI[0])
    for c in _ACOS4_PI[1:]:
        p = p * x + jnp.float32(c)
    return p


def _ag_kernel(a_ref, g_ref, b_ref, ab_ref):
    a = a_ref[...]
    b_ref[...] = jax.lax.dot_general(
        a, g_ref[...], (((1,), (0,)), ((), ())),
        preferred_element_type=jnp.float32)
    ab_ref[...] = a.astype(ab_ref.dtype)


def _theta_math(s0, s):
    ax = jnp.abs(s)
    t = 1.0 - ax
    rp = (t * jax.lax.rsqrt(t)) * _acospi_poly(ax)
    w = jnp.where(s >= 0, 1.0 - rp, rp)
    u = t * (1.0 + ax)
    sq1p = (u * jax.lax.rsqrt(u)) * jnp.float32(_INV_PI)
    k1 = s * w + sq1p
    t2 = 1.0 - k1
    degree = 1.0 - (t2 * jax.lax.rsqrt(t2)) * _acospi_poly(k1)
    return s0 * degree + k1


def _theta_kernel(b1_ref, b2_ref, o_ref, s0_ref, d1_ref):
    bm = b1_ref.shape[0]
    bn = b2_ref.shape[0]
    b1 = b1_ref[...]
    b2 = b2_ref[...]
    n1 = jnp.sum(b1 * b1, axis=1, keepdims=True)
    r1 = jax.lax.rsqrt(n1)
    n2 = jnp.sum(b2 * b2, axis=1, keepdims=True)
    r2 = jax.lax.rsqrt(n2)
    s0_ref[...] = jax.lax.dot_general(
        b1 * r1, b2 * r2, (((1,), (1,)), ((), ())),
        preferred_element_type=jnp.float32)
    d1_ref[...] = jnp.broadcast_to(n1 * r1, (bm, 128))
    d2x = jnp.broadcast_to((n2 * r2).T, (16, bn))

    def body(i, carry):
        sl = pl.ds(i * 16, 16)
        s0 = s0_ref[sl, :]
        s = jnp.clip(s0, -0.9999, 0.9999)
        val = _theta_math(s0, s)
        d1c = d1_ref[sl, 0:1]
        o_ref[sl, :] = ((val * d1c) * d2x).astype(o_ref.dtype)
        return carry

    jax.lax.fori_loop(0, bm // 16, body, 0, unroll=1)


def _mm_kernel(x_ref, y_ref, o_ref, *, trans_y):
    dn = (((1,), (1 if trans_y else 0,)), ((), ()))
    o_ref[...] = jax.lax.dot_general(
        x_ref[...], y_ref[...], dn,
        preferred_element_type=jnp.float32).astype(o_ref.dtype)


def _matmul(x, y, trans_y, bm, bn, out_dtype):
    M, K = x.shape
    N = y.shape[0] if trans_y else y.shape[1]
    if trans_y:
        y_spec = pl.BlockSpec((bn, K), lambda m, n: (n, 0))
    else:
        y_spec = pl.BlockSpec((K, bn), lambda m, n: (0, n))
    return pl.pallas_call(
        functools.partial(_mm_kernel, trans_y=trans_y),
        grid=(M // bm, N // bn),
        in_specs=[pl.BlockSpec((bm, K), lambda m, n: (m, 0)), y_spec],
        out_specs=pl.BlockSpec((bm, bn), lambda m, n: (m, n)),
        out_shape=jax.ShapeDtypeStruct((M, N), out_dtype),
        compiler_params=pltpu.CompilerParams(
            dimension_semantics=("parallel", "parallel")),
    )(x, y)


def _a_times_g(A, g, bm):
    M, K = A.shape
    D = g.shape[1]
    return pl.pallas_call(
        _ag_kernel,
        grid=(M // bm,),
        in_specs=[
            pl.BlockSpec((bm, K), lambda m: (m, 0)),
            pl.BlockSpec((K, D), lambda m: (0, 0)),
        ],
        out_specs=[
            pl.BlockSpec((bm, D), lambda m: (m, 0)),
            pl.BlockSpec((bm, K), lambda m: (m, 0)),
        ],
        out_shape=[
            jax.ShapeDtypeStruct((M, D), jnp.float32),
            jax.ShapeDtypeStruct((M, K), jnp.bfloat16),
        ],
        compiler_params=pltpu.CompilerParams(
            dimension_semantics=("parallel",)),
    )(A, g)


def _theta(B1, B2, bm, bn, out_dtype):
    M = B1.shape[0]
    N = B2.shape[0]
    D = B1.shape[1]
    return pl.pallas_call(
        _theta_kernel,
        grid=(M // bm, N // bn),
        in_specs=[
            pl.BlockSpec((bm, D), lambda m, n: (m, 0)),
            pl.BlockSpec((bn, D), lambda m, n: (n, 0)),
        ],
        out_specs=pl.BlockSpec((bm, bn), lambda m, n: (m, n)),
        out_shape=jax.ShapeDtypeStruct((M, N), out_dtype),
        scratch_shapes=[
            pltpu.VMEM((bm, bn), jnp.float32),
            pltpu.VMEM((bm, 128), jnp.float32),
        ],
        compiler_params=pltpu.CompilerParams(
            dimension_semantics=("parallel", "parallel")),
    )(B1, B2)


def kernel(g1, g2, A1, A2):
    B1, A1b = _a_times_g(A1, g1, bm=512)
    B2, A2b = _a_times_g(A2, g2, bm=512)
    theta = _theta(B1, B2, bm=512, bn=512, out_dtype=jnp.bfloat16)
    T = _matmul(A1b, theta, trans_y=False, bm=1024, bn=1024,
                out_dtype=jnp.bfloat16)
    out = _matmul(T, A2b, trans_y=True, bm=1024, bn=1024,
                  out_dtype=jnp.float32)
    return out


# speedup vs baseline: 1.1990x; 1.1990x over previous
import functools
import math

import jax
import jax.numpy as jnp
from jax.experimental import pallas as pl
from jax.experimental.pallas import tpu as pltpu

_PI = math.pi

_ACOS4_PI = tuple(
    c / _PI for c in (-0.0187293, 0.0742610, -0.2121144, 1.5707288))
_INV_PI = 1.0 / _PI


def _acospi_poly(x):
    p = jnp.float32(_ACOS4_PI[0])
    for c in _ACOS4_PI[1:]:
        p = p * x + jnp.float32(c)
    return p


def _ag_kernel(a_ref, g_ref, b_ref, ab_ref):
    a = a_ref[...]
    b_ref[...] = jax.lax.dot_general(
        a, g_ref[...], (((1,), (0,)), ((), ())),
        preferred_element_type=jnp.float32)
    ab_ref[...] = a.astype(ab_ref.dtype)


def _theta_math(s0, s):
    ax = jnp.abs(s)
    t = 1.0 - ax
    rp = (t * jax.lax.rsqrt(t)) * _acospi_poly(ax)
    w = jnp.where(s >= 0, 1.0 - rp, rp)
    u = t * (1.0 + ax)
    sq1p = (u * jax.lax.rsqrt(u)) * jnp.float32(_INV_PI)
    k1 = s * w + sq1p
    t2 = 1.0 - k1
    degree = 1.0 - (t2 * jax.lax.rsqrt(t2)) * _acospi_poly(k1)
    return s0 * degree + k1


def _theta_kernel(b1_ref, b2_ref, o_ref):
    b1 = b1_ref[...]
    b2 = b2_ref[...]
    n1 = jnp.sum(b1 * b1, axis=1, keepdims=True)
    r1 = jax.lax.rsqrt(n1)
    n2 = jnp.sum(b2 * b2, axis=1, keepdims=True)
    r2 = jax.lax.rsqrt(n2)
    s0 = jax.lax.dot_general(
        b1 * r1, b2 * r2, (((1,), (1,)), ((), ())),
        preferred_element_type=jnp.float32)
    s = jnp.clip(s0, -0.9999, 0.9999)
    val = _theta_math(s0, s)
    d1 = n1 * r1
    d2t = (n2 * r2).T
    o_ref[...] = ((val * d1) * d2t).astype(o_ref.dtype)


def _mm_kernel(x_ref, y_ref, o_ref, *, trans_y):
    dn = (((1,), (1 if trans_y else 0,)), ((), ()))
    o_ref[...] = jax.lax.dot_general(
        x_ref[...], y_ref[...], dn,
        preferred_element_type=jnp.float32).astype(o_ref.dtype)


def _matmul(x, y, trans_y, bm, bn, out_dtype):
    M, K = x.shape
    N = y.shape[0] if trans_y else y.shape[1]
    if trans_y:
        y_spec = pl.BlockSpec((bn, K), lambda m, n: (n, 0))
    else:
        y_spec = pl.BlockSpec((K, bn), lambda m, n: (0, n))
    return pl.pallas_call(
        functools.partial(_mm_kernel, trans_y=trans_y),
        grid=(M // bm, N // bn),
        in_specs=[pl.BlockSpec((bm, K), lambda m, n: (m, 0)), y_spec],
        out_specs=pl.BlockSpec((bm, bn), lambda m, n: (m, n)),
        out_shape=jax.ShapeDtypeStruct((M, N), out_dtype),
        compiler_params=pltpu.CompilerParams(
            dimension_semantics=("parallel", "parallel")),
    )(x, y)


def _a_times_g(A, g, bm):
    M, K = A.shape
    D = g.shape[1]
    return pl.pallas_call(
        _ag_kernel,
        grid=(M // bm,),
        in_specs=[
            pl.BlockSpec((bm, K), lambda m: (m, 0)),
            pl.BlockSpec((K, D), lambda m: (0, 0)),
        ],
        out_specs=[
            pl.BlockSpec((bm, D), lambda m: (m, 0)),
            pl.BlockSpec((bm, K), lambda m: (m, 0)),
        ],
        out_shape=[
            jax.ShapeDtypeStruct((M, D), jnp.float32),
            jax.ShapeDtypeStruct((M, K), jnp.bfloat16),
        ],
        compiler_params=pltpu.CompilerParams(
            dimension_semantics=("parallel",)),
    )(A, g)


def _theta(B1, B2, bm, bn, out_dtype):
    M = B1.shape[0]
    N = B2.shape[0]
    D = B1.shape[1]
    return pl.pallas_call(
        _theta_kernel,
        grid=(M // bm, N // bn),
        in_specs=[
            pl.BlockSpec((bm, D), lambda m, n: (m, 0)),
            pl.BlockSpec((bn, D), lambda m, n: (n, 0)),
        ],
        out_specs=pl.BlockSpec((bm, bn), lambda m, n: (m, n)),
        out_shape=jax.ShapeDtypeStruct((M, N), out_dtype),
        compiler_params=pltpu.CompilerParams(
            dimension_semantics=("parallel", "parallel")),
    )(B1, B2)


def kernel(g1, g2, A1, A2):
    B1, A1b = _a_times_g(A1, g1, bm=512)
    B2, A2b = _a_times_g(A2, g2, bm=512)
    theta = _theta(B1, B2, bm=512, bn=512, out_dtype=jnp.bfloat16)
    T = _matmul(A1b, theta, trans_y=False, bm=1024, bn=1024,
                out_dtype=jnp.bfloat16)
    out = _matmul(T, A2b, trans_y=True, bm=1024, bn=1024,
                  out_dtype=jnp.float32)
    return out


# speedup vs baseline: 1.4941x; 1.2461x over previous
import math

import jax
import jax.numpy as jnp
from jax.experimental import pallas as pl
from jax.experimental.pallas import tpu as pltpu

_PI = math.pi

_ACOS4_PI = tuple(
    c / _PI for c in (-0.0187293, 0.0742610, -0.2121144, 1.5707288))
_INV_PI = 1.0 / _PI

_N = 2048
_D = 128
_BR = 512
_NB = _N // _BR

_DNN = (((1,), (0,)), ((), ()))
_DNT = (((1,), (1,)), ((), ()))


def _acospi_poly(x):
    p = jnp.float32(_ACOS4_PI[0])
    for c in _ACOS4_PI[1:]:
        p = p * x + jnp.float32(c)
    return p


def _theta_math(s0, s):
    ax = jnp.abs(s)
    t = 1.0 - ax
    rp = (t * jax.lax.rsqrt(t)) * _acospi_poly(ax)
    w = jnp.where(s >= 0, 1.0 - rp, rp)
    u = t * (1.0 + ax)
    sq1p = (u * jax.lax.rsqrt(u)) * jnp.float32(_INV_PI)
    k1 = s * w + sq1p
    t2 = 1.0 - k1
    degree = 1.0 - (t2 * jax.lax.rsqrt(t2)) * _acospi_poly(k1)
    return s0 * degree + k1


def _stage1(a, g):
    b = jax.lax.dot_general(a, g, _DNN, preferred_element_type=jnp.float32)
    n = jnp.sum(b * b, axis=1, keepdims=True)
    r = jax.lax.rsqrt(n)
    return b * r, n * r, a.astype(jnp.bfloat16)


def _mega_kernel(a1_ref, g1_ref, a2_ref, g2_ref, o_ref,
                 a1b_ref, a2b_ref, b1n_ref, b2n_ref, d1_ref, d2t_ref,
                 th_ref):
    p = pl.program_id(0)

    @pl.when(p < _NB)
    def _():
        rows = pl.ds((p % _NB) * _BR, _BR)
        bn, d, ab = _stage1(a1_ref[...], g1_ref[...])
        b1n_ref[rows, :] = bn
        d1_ref[rows, :] = jnp.broadcast_to(d, (_BR, 128))
        a1b_ref[rows, :] = ab

    @pl.when((p >= _NB) & (p < 2 * _NB))
    def _():
        cols = pl.ds((p % _NB) * _BR, _BR)
        rows = pl.ds((p % _NB) * _BR, _BR)
        bn, d, ab = _stage1(a2_ref[...], g2_ref[...])
        b2n_ref[rows, :] = bn
        d2t_ref[:, cols] = jnp.broadcast_to(d.T, (8, _BR))
        a2b_ref[rows, :] = ab

    @pl.when((p >= 2 * _NB) & (p < 3 * _NB))
    def _():
        rows = pl.ds((p % _NB) * _BR, _BR)
        s0 = jax.lax.dot_general(
            b1n_ref[rows, :], b2n_ref[...], _DNT,
            preferred_element_type=jnp.float32)
        s = jnp.clip(s0, -0.9999, 0.9999)
        val = _theta_math(s0, s)
        d1c = d1_ref[rows, 0:1]
        d2t = d2t_ref[0:1, :]
        th_ref[rows, :] = ((val * d1c) * d2t).astype(th_ref.dtype)

    @pl.when((p >= 3 * _NB) & (p < 4 * _NB))
    def _():
        rows = pl.ds((p % _NB) * _BR, _BR)
        t = jax.lax.dot_general(
            a1b_ref[rows, :], th_ref[...], _DNN,
            preferred_element_type=jnp.float32)
        a1b_ref[rows, :] = t.astype(a1b_ref.dtype)

    @pl.when(p >= 4 * _NB)
    def _():
        rows = pl.ds((p % _NB) * _BR, _BR)
        o_ref[...] = jax.lax.dot_general(
            a1b_ref[rows, :], a2b_ref[...], _DNT,
            preferred_element_type=jnp.float32)


def kernel(g1, g2, A1, A2):
    nsteps = 5 * _NB

    def a1_map(p):
        return (jnp.clip(p, 0, _NB - 1), 0)

    def a2_map(p):
        return (jnp.clip(p - _NB, 0, _NB - 1), 0)

    def o_map(p):
        return (jnp.clip(p - 4 * _NB, 0, _NB - 1), 0)

    return pl.pallas_call(
        _mega_kernel,
        grid=(nsteps,),
        in_specs=[
            pl.BlockSpec((_BR, _N), a1_map),
            pl.BlockSpec((_N, _D), lambda p: (0, 0)),
            pl.BlockSpec((_BR, _N), a2_map),
            pl.BlockSpec((_N, _D), lambda p: (0, 0)),
        ],
        out_specs=pl.BlockSpec((_BR, _N), o_map),
        out_shape=jax.ShapeDtypeStruct((_N, _N), jnp.float32),
        scratch_shapes=[
            pltpu.VMEM((_N, _N), jnp.bfloat16),
            pltpu.VMEM((_N, _N), jnp.bfloat16),
            pltpu.VMEM((_N, _D), jnp.float32),
            pltpu.VMEM((_N, _D), jnp.float32),
            pltpu.VMEM((_N, 128), jnp.float32),
            pltpu.VMEM((8, _N), jnp.float32),
            pltpu.VMEM((_N, _N), jnp.bfloat16),
        ],
        compiler_params=pltpu.CompilerParams(
            dimension_semantics=("arbitrary",)),
    )(A1, g1, A2, g2)
